# baseline (speedup 1.0000x reference)
import jax
import jax.numpy as jnp
from jax.experimental import pallas as pl

_LANES = 512
_ROWS = 8
_E = _LANES * _ROWS


def _body(x_ref, g_ref, w1_ref, b1_ref, out_ref):
    g = g_ref[...]
    w1 = w1_ref[...]
    b1 = b1_ref[...]
    x = x_ref[...]
    for r in range(_ROWS):
        xr = x[r:r + 1, :]
        h = jnp.maximum(w1 * xr + b1, 0.0)
        pq = jax.lax.dot_general(g, h, (((1,), (0,)), ((), ())),
                                 preferred_element_type=jnp.float32)
        t = pq.T
        out_ref[r * _LANES:(r + 1) * _LANES, :] = t[:, :4]


def kernel(rx, phase, W1, b1, W2, b2):
    del phase
    n = rx.shape[0]
    n_states = W2.shape[1]
    hidden = W1.shape[1]
    nblocks = n // _E

    w1e = jnp.zeros((128, 1), jnp.float32).at[:hidden, 0].set(W1[0, :])
    b1e = jnp.zeros((128, 1), jnp.float32).at[:hidden, 0].set(b1)
    b1e = b1e.at[hidden, 0].set(1.0)
    g = jnp.zeros((8, 128), jnp.float32).at[:n_states, :hidden].set(W2.T)
    g = g.at[:n_states, hidden].set(b2)

    x = rx.reshape(nblocks * _ROWS, _LANES)

    out = pl.pallas_call(
        _body,
        grid=(nblocks,),
        in_specs=[
            pl.BlockSpec((_ROWS, _LANES), lambda i: (i, 0)),
            pl.BlockSpec((8, 128), lambda i: (0, 0)),
            pl.BlockSpec((128, 1), lambda i: (0, 0)),
            pl.BlockSpec((128, 1), lambda i: (0, 0)),
        ],
        out_specs=pl.BlockSpec((_E, 4), lambda i: (i, 0)),
        out_shape=jax.ShapeDtypeStruct((n, 4), jnp.float32),
    )(x, g, w1e, b1e)
    return out
